# Initial kernel scaffold; baseline (speedup 1.0000x reference)
import jax
import jax.numpy as jnp
from jax.experimental import pallas as pl


def kernel(unary, binary, deltas, index1, index2):
    raise NotImplementedError("write your pallas kernel here")


# speedup vs baseline: 9.4442x; 9.4442x over previous
import functools

import jax
import jax.numpy as jnp
from jax import lax
from jax.experimental import pallas as pl
from jax.experimental.pallas import tpu as pltpu
from jax.experimental.pallas import tpu_sc as plsc

_E = 320000
_N = 10000
_NU = 128
_NB = 16

_NC = 2
_NS = 16
_NW = _NC * _NS
_L = 128
_CPW = -(-(_E // _L) // _NW)
_EPAD = _NW * _CPW * _L


def _sc_scatter_body(idx1, idx2, vx, vy, zinit, part0, part1,
                     idx_v, val_v, acc):
    c = lax.axis_index("c")
    s = lax.axis_index("s")
    w = s * _NC + c

    @pl.when(s == 0)
    def _zero():
        pltpu.sync_copy(zinit, acc)

    plsc.subcore_barrier()

    def scatter_slab(idx_hbm, val_hbm):
        pltpu.sync_copy(idx_hbm.at[w], idx_v)
        pltpu.sync_copy(val_hbm.at[w], val_v)

        def body(j, carry):
            pltpu.sync_copy(val_v.at[j], acc.at[idx_v.at[j]], add=True)
            return carry

        lax.fori_loop(0, _CPW, body, 0)

    scatter_slab(idx1, vx)
    scatter_slab(idx2, vy)

    plsc.subcore_barrier()

    @pl.when((s == 0) & (c == 0))
    def _flush0():
        pltpu.sync_copy(acc, part0)

    @pl.when((s == 0) & (c == 1))
    def _flush1():
        pltpu.sync_copy(acc, part1)


_sc_scatter = functools.partial(
    pl.kernel,
    out_type=(
        jax.ShapeDtypeStruct((_N,), jnp.float32),
        jax.ShapeDtypeStruct((_N,), jnp.float32),
    ),
    mesh=plsc.VectorSubcoreMesh(core_axis_name="c", subcore_axis_name="s",
                                num_cores=_NC, num_subcores=_NS),
    scratch_types=[
        pltpu.VMEM((_CPW, _L), jnp.int32),
        pltpu.VMEM((_CPW, _L), jnp.float32),
        pltpu.VMEM_SHARED((_N,), jnp.float32),
    ],
    compiler_params=pltpu.CompilerParams(use_tc_tiling_on_sc=False),
)(_sc_scatter_body)


def _tc_assemble_body(part_ref, out_ref):
    onehot = (lax.broadcasted_iota(jnp.int32, (_NC, _NU), 1) == 0)
    out_ref[...] = lax.dot_general(
        part_ref[...],
        onehot.astype(jnp.float32),
        (((0,), (0,)), ((), ())),
        preferred_element_type=jnp.float32,
        precision=lax.Precision.HIGHEST,
    )


def kernel(unary, binary, deltas, index1, index2):
    del unary, binary

    pad = _EPAD - _E
    i1 = jnp.pad(index1.astype(jnp.int32), (0, pad)).reshape(_NW, _CPW, _L)
    i2 = jnp.pad(index2.astype(jnp.int32), (0, pad)).reshape(_NW, _CPW, _L)
    vx = jnp.pad(deltas[:, 0], (0, pad)).reshape(_NW, _CPW, _L)
    vy = jnp.pad(deltas[:, _NU], (0, pad)).reshape(_NW, _CPW, _L)
    zinit = jnp.zeros((_N,), jnp.float32)

    p0, p1 = _sc_scatter(i1, i2, vx, vy, zinit)
    partials = jnp.stack([p0, p1])

    out1 = pl.pallas_call(
        _tc_assemble_body,
        in_specs=[pl.BlockSpec((_NC, _N), lambda: (0, 0))],
        out_specs=pl.BlockSpec((_N, _NU), lambda: (0, 0)),
        out_shape=jax.ShapeDtypeStruct((_N, _NU), jnp.float32),
    )(partials)

    return (out1, deltas[:, 2 * _NU:])


# speedup vs baseline: 9.7196x; 1.0292x over previous
import functools

import jax
import jax.numpy as jnp
from jax import lax
from jax.experimental import pallas as pl
from jax.experimental.pallas import tpu as pltpu
from jax.experimental.pallas import tpu_sc as plsc

_E = 320000
_N = 10000
_NU = 128
_NB = 16

_NC = 2
_NS = 16
_NW = _NC * _NS
_L = 128
_CPW = -(-(_E // _L) // _NW)
_EPAD = _NW * _CPW * _L


_WINDOW = 16


def _sc_scatter_body(idx1, idx2, vx, vy, zinit, part0, part1,
                     i1_v, i2_v, vx_v, vy_v, acc, lsem, ssem):
    c = lax.axis_index("c")
    s = lax.axis_index("s")
    w = s * _NC + c

    @pl.when(s == 0)
    def _zero():
        pltpu.sync_copy(zinit, acc)

    loads = [
        pltpu.async_copy(idx1.at[w], i1_v, lsem),
        pltpu.async_copy(idx2.at[w], i2_v, lsem),
        pltpu.async_copy(vx.at[w], vx_v, lsem),
        pltpu.async_copy(vy.at[w], vy_v, lsem),
    ]
    for d in loads:
        d.wait()

    plsc.subcore_barrier()

    pairs = [(vx_v, i1_v), (vy_v, i2_v)]
    descs = []
    for val_v, idx_v in pairs:
        for j in range(_CPW):
            if len(descs) >= _WINDOW:
                descs[len(descs) - _WINDOW].wait()
            descs.append(
                pltpu.async_copy(val_v.at[j], acc.at[idx_v.at[j]], ssem,
                                 add=True))
    for d in descs[-_WINDOW:]:
        d.wait()

    plsc.subcore_barrier()

    @pl.when((s == 0) & (c == 0))
    def _flush0():
        pltpu.sync_copy(acc, part0)

    @pl.when((s == 0) & (c == 1))
    def _flush1():
        pltpu.sync_copy(acc, part1)


_sc_scatter = functools.partial(
    pl.kernel,
    out_type=(
        jax.ShapeDtypeStruct((_N,), jnp.float32),
        jax.ShapeDtypeStruct((_N,), jnp.float32),
    ),
    mesh=plsc.VectorSubcoreMesh(core_axis_name="c", subcore_axis_name="s",
                                num_cores=_NC, num_subcores=_NS),
    scratch_types=[
        pltpu.VMEM((_CPW, _L), jnp.int32),
        pltpu.VMEM((_CPW, _L), jnp.int32),
        pltpu.VMEM((_CPW, _L), jnp.float32),
        pltpu.VMEM((_CPW, _L), jnp.float32),
        pltpu.VMEM_SHARED((_N,), jnp.float32),
        pltpu.SemaphoreType.DMA,
        pltpu.SemaphoreType.DMA,
    ],
    compiler_params=pltpu.CompilerParams(use_tc_tiling_on_sc=False),
)(_sc_scatter_body)


def _tc_assemble_body(part_ref, out_ref):
    onehot = (lax.broadcasted_iota(jnp.int32, (_NC, _NU), 1) == 0)
    out_ref[...] = lax.dot_general(
        part_ref[...],
        onehot.astype(jnp.float32),
        (((0,), (0,)), ((), ())),
        preferred_element_type=jnp.float32,
        precision=lax.Precision.HIGHEST,
    )


def kernel(unary, binary, deltas, index1, index2):
    del unary, binary

    pad = _EPAD - _E
    i1 = jnp.pad(index1.astype(jnp.int32), (0, pad)).reshape(_NW, _CPW, _L)
    i2 = jnp.pad(index2.astype(jnp.int32), (0, pad)).reshape(_NW, _CPW, _L)
    vx = jnp.pad(deltas[:, 0], (0, pad)).reshape(_NW, _CPW, _L)
    vy = jnp.pad(deltas[:, _NU], (0, pad)).reshape(_NW, _CPW, _L)
    zinit = jnp.zeros((_N,), jnp.float32)

    p0, p1 = _sc_scatter(i1, i2, vx, vy, zinit)
    partials = jnp.stack([p0, p1])

    out1 = pl.pallas_call(
        _tc_assemble_body,
        in_specs=[pl.BlockSpec((_NC, _N), lambda: (0, 0))],
        out_specs=pl.BlockSpec((_N, _NU), lambda: (0, 0)),
        out_shape=jax.ShapeDtypeStruct((_N, _NU), jnp.float32),
    )(partials)

    return (out1, deltas[:, 2 * _NU:])


# speedup vs baseline: 10.1137x; 1.0405x over previous
import functools

import jax
import jax.numpy as jnp
from jax import lax
from jax.experimental import pallas as pl
from jax.experimental.pallas import tpu as pltpu
from jax.experimental.pallas import tpu_sc as plsc

_E = 320000
_N = 10000
_NU = 128
_NB = 16

_NC = 2
_NS = 16
_NW = _NC * _NS
_L = 128
_CPW = -(-(_E // _L) // _NW)
_EPAD = _NW * _CPW * _L


_WINDOW = 16


def _sc_scatter_body(idx1, idx2, vx, vy, zinit, part0, part1,
                     i1_v, i2_v, vx_v, vy_v, acc, lsem, ssem):
    c = lax.axis_index("c")
    s = lax.axis_index("s")
    w = s * _NC + c

    @pl.when(s == 0)
    def _zero():
        pltpu.sync_copy(zinit, acc)

    loads = [
        pltpu.async_copy(idx1.at[w], i1_v, lsem),
        pltpu.async_copy(idx2.at[w], i2_v, lsem),
        pltpu.async_copy(vx.at[w], vx_v, lsem),
        pltpu.async_copy(vy.at[w], vy_v, lsem),
    ]
    for d in loads:
        d.wait()

    plsc.subcore_barrier()

    d1 = pltpu.async_copy(vx_v, acc.at[i1_v], ssem, add=True)
    d2 = pltpu.async_copy(vy_v, acc.at[i2_v], ssem, add=True)
    d1.wait()
    d2.wait()

    plsc.subcore_barrier()

    @pl.when((s == 0) & (c == 0))
    def _flush0():
        pltpu.sync_copy(acc, part0)

    @pl.when((s == 0) & (c == 1))
    def _flush1():
        pltpu.sync_copy(acc, part1)


_sc_scatter = functools.partial(
    pl.kernel,
    out_type=(
        jax.ShapeDtypeStruct((_N,), jnp.float32),
        jax.ShapeDtypeStruct((_N,), jnp.float32),
    ),
    mesh=plsc.VectorSubcoreMesh(core_axis_name="c", subcore_axis_name="s",
                                num_cores=_NC, num_subcores=_NS),
    scratch_types=[
        pltpu.VMEM((_CPW * _L,), jnp.int32),
        pltpu.VMEM((_CPW * _L,), jnp.int32),
        pltpu.VMEM((_CPW * _L,), jnp.float32),
        pltpu.VMEM((_CPW * _L,), jnp.float32),
        pltpu.VMEM_SHARED((_N,), jnp.float32),
        pltpu.SemaphoreType.DMA,
        pltpu.SemaphoreType.DMA,
    ],
    compiler_params=pltpu.CompilerParams(use_tc_tiling_on_sc=False),
)(_sc_scatter_body)


def _tc_assemble_body(part_ref, out_ref):
    onehot = (lax.broadcasted_iota(jnp.int32, (_NC, _NU), 1) == 0)
    out_ref[...] = lax.dot_general(
        part_ref[...],
        onehot.astype(jnp.float32),
        (((0,), (0,)), ((), ())),
        preferred_element_type=jnp.float32,
        precision=lax.Precision.HIGHEST,
    )


def kernel(unary, binary, deltas, index1, index2):
    del unary, binary

    pad = _EPAD - _E
    i1 = jnp.pad(index1.astype(jnp.int32), (0, pad)).reshape(_NW, _CPW * _L)
    i2 = jnp.pad(index2.astype(jnp.int32), (0, pad)).reshape(_NW, _CPW * _L)
    vx = jnp.pad(deltas[:, 0], (0, pad)).reshape(_NW, _CPW * _L)
    vy = jnp.pad(deltas[:, _NU], (0, pad)).reshape(_NW, _CPW * _L)
    zinit = jnp.zeros((_N,), jnp.float32)

    p0, p1 = _sc_scatter(i1, i2, vx, vy, zinit)
    partials = jnp.stack([p0, p1])

    out1 = pl.pallas_call(
        _tc_assemble_body,
        in_specs=[pl.BlockSpec((_NC, _N), lambda: (0, 0))],
        out_specs=pl.BlockSpec((_N, _NU), lambda: (0, 0)),
        out_shape=jax.ShapeDtypeStruct((_N, _NU), jnp.float32),
    )(partials)

    return (out1, deltas[:, 2 * _NU:])


# speedup vs baseline: 12.1125x; 1.1976x over previous
import functools

import jax
import jax.numpy as jnp
from jax import lax
from jax.experimental import pallas as pl
from jax.experimental.pallas import tpu as pltpu
from jax.experimental.pallas import tpu_sc as plsc

_E = 320000
_N = 10000
_NU = 128
_NB = 16

_NC = 2
_NS = 16
_NW = _NC * _NS
_EPW = _E // _NW

_BN = 2000
_NBLK = _N // _BN


def _sc_scatter_body(idx1, idx2, vx, vy, zinit, part0, part1,
                     i1_v, i2_v, vx_v, vy_v, acc, lsem, ssem):
    c = lax.axis_index("c")
    s = lax.axis_index("s")
    w = s * _NC + c
    base = w * _EPW

    @pl.when(s == 0)
    def _zero():
        pltpu.sync_copy(zinit, acc)

    loads = [
        pltpu.async_copy(idx1.at[pl.ds(base, _EPW)], i1_v, lsem),
        pltpu.async_copy(idx2.at[pl.ds(base, _EPW)], i2_v, lsem),
        pltpu.async_copy(vx.at[pl.ds(base, _EPW)], vx_v, lsem),
        pltpu.async_copy(vy.at[pl.ds(base, _EPW)], vy_v, lsem),
    ]
    for d in loads:
        d.wait()

    plsc.subcore_barrier()

    d1 = pltpu.async_copy(vx_v, acc.at[i1_v], ssem, add=True)
    d2 = pltpu.async_copy(vy_v, acc.at[i2_v], ssem, add=True)
    d1.wait()
    d2.wait()

    plsc.subcore_barrier()

    @pl.when((s == 0) & (c == 0))
    def _flush0():
        pltpu.sync_copy(acc, part0)

    @pl.when((s == 0) & (c == 1))
    def _flush1():
        pltpu.sync_copy(acc, part1)


_sc_scatter = functools.partial(
    pl.kernel,
    out_type=(
        jax.ShapeDtypeStruct((_N,), jnp.float32),
        jax.ShapeDtypeStruct((_N,), jnp.float32),
    ),
    mesh=plsc.VectorSubcoreMesh(core_axis_name="c", subcore_axis_name="s",
                                num_cores=_NC, num_subcores=_NS),
    scratch_types=[
        pltpu.VMEM((_EPW,), jnp.int32),
        pltpu.VMEM((_EPW,), jnp.int32),
        pltpu.VMEM((_EPW,), jnp.float32),
        pltpu.VMEM((_EPW,), jnp.float32),
        pltpu.VMEM_SHARED((_N,), jnp.float32),
        pltpu.SemaphoreType.DMA,
        pltpu.SemaphoreType.DMA,
    ],
    compiler_params=pltpu.CompilerParams(use_tc_tiling_on_sc=False),
)(_sc_scatter_body)


def _tc_assemble_body(part_ref, out_ref):
    onehot = (lax.broadcasted_iota(jnp.int32, (_NC, _NU), 1) == 0)
    out_ref[...] = lax.dot_general(
        part_ref[0],
        onehot.astype(jnp.float32),
        (((0,), (0,)), ((), ())),
        preferred_element_type=jnp.float32,
        precision=lax.Precision.HIGHEST,
    )


def kernel(unary, binary, deltas, index1, index2):
    del unary, binary

    i1 = index1.astype(jnp.int32)
    i2 = index2.astype(jnp.int32)
    vx = deltas[:, 0]
    vy = deltas[:, _NU]
    zinit = jnp.zeros((_N,), jnp.float32)

    p0, p1 = _sc_scatter(i1, i2, vx, vy, zinit)
    parts = jnp.stack([p0, p1]).reshape(_NC, _NBLK, _BN).transpose(1, 0, 2)

    out1 = pl.pallas_call(
        _tc_assemble_body,
        grid=(_NBLK,),
        in_specs=[pl.BlockSpec((1, _NC, _BN), lambda i: (i, 0, 0))],
        out_specs=pl.BlockSpec((_BN, _NU), lambda i: (i, 0)),
        out_shape=jax.ShapeDtypeStruct((_N, _NU), jnp.float32),
    )(parts)

    return (out1, deltas[:, 2 * _NU:])


# speedup vs baseline: 12.3024x; 1.0157x over previous
import functools

import jax
import jax.numpy as jnp
from jax import lax
from jax.experimental import pallas as pl
from jax.experimental.pallas import tpu as pltpu
from jax.experimental.pallas import tpu_sc as plsc

_E = 320000
_N = 10000
_NU = 128
_NB = 16

_NC = 2
_NS = 16
_NW = _NC * _NS
_EPW = _E // _NW

_BN = 2000
_NBLK = _N // _BN


def _sc_scatter_body(idx1, idx2, vx, vy, zinit, part,
                     i1_v, i2_v, vx_v, vy_v, acc, lsem, ssem):
    c = lax.axis_index("c")
    s = lax.axis_index("s")
    w = s * _NC + c
    base = w * _EPW

    @pl.when(s == 0)
    def _zero():
        pltpu.sync_copy(zinit, acc)

    loads = [
        pltpu.async_copy(idx1.at[pl.ds(base, _EPW)], i1_v, lsem),
        pltpu.async_copy(idx2.at[pl.ds(base, _EPW)], i2_v, lsem),
        pltpu.async_copy(vx.at[pl.ds(base, _EPW)], vx_v, lsem),
        pltpu.async_copy(vy.at[pl.ds(base, _EPW)], vy_v, lsem),
    ]
    for d in loads:
        d.wait()

    plsc.subcore_barrier()

    d1 = pltpu.async_copy(vx_v, acc.at[i1_v], ssem, add=True)
    d2 = pltpu.async_copy(vy_v, acc.at[i2_v], ssem, add=True)
    d1.wait()
    d2.wait()

    plsc.subcore_barrier()

    @pl.when(s < _NBLK)
    def _flush():
        pltpu.sync_copy(acc.at[pl.ds(s * _BN, _BN)], part.at[c, s, 0])


_sc_scatter = functools.partial(
    pl.kernel,
    out_type=jax.ShapeDtypeStruct((_NC, _NBLK, 1, _BN), jnp.float32),
    mesh=plsc.VectorSubcoreMesh(core_axis_name="c", subcore_axis_name="s",
                                num_cores=_NC, num_subcores=_NS),
    scratch_types=[
        pltpu.VMEM((_EPW,), jnp.int32),
        pltpu.VMEM((_EPW,), jnp.int32),
        pltpu.VMEM((_EPW,), jnp.float32),
        pltpu.VMEM((_EPW,), jnp.float32),
        pltpu.VMEM_SHARED((_N,), jnp.float32),
        pltpu.SemaphoreType.DMA,
        pltpu.SemaphoreType.DMA,
    ],
    compiler_params=pltpu.CompilerParams(use_tc_tiling_on_sc=False),
)(_sc_scatter_body)


def _tc_assemble_body(part_ref, out_ref):
    onehot = (lax.broadcasted_iota(jnp.int32, (_NC, _NU), 1) == 0)
    out_ref[...] = lax.dot_general(
        part_ref[:, 0, 0, :],
        onehot.astype(jnp.float32),
        (((0,), (0,)), ((), ())),
        preferred_element_type=jnp.float32,
        precision=lax.Precision.HIGHEST,
    )


def kernel(unary, binary, deltas, index1, index2):
    del unary, binary

    i1 = index1.astype(jnp.int32)
    i2 = index2.astype(jnp.int32)
    vx = deltas[:, 0]
    vy = deltas[:, _NU]
    zinit = jnp.zeros((_N,), jnp.float32)

    parts = _sc_scatter(i1, i2, vx, vy, zinit)

    out1 = pl.pallas_call(
        _tc_assemble_body,
        grid=(_NBLK,),
        in_specs=[pl.BlockSpec((_NC, 1, 1, _BN), lambda i: (0, i, 0, 0))],
        out_specs=pl.BlockSpec((_BN, _NU), lambda i: (i, 0)),
        out_shape=jax.ShapeDtypeStruct((_N, _NU), jnp.float32),
    )(parts)

    return (out1, deltas[:, 2 * _NU:])


# speedup vs baseline: 12.8843x; 1.0473x over previous
import functools

import jax
import jax.numpy as jnp
from jax import lax
from jax.experimental import pallas as pl
from jax.experimental.pallas import tpu as pltpu
from jax.experimental.pallas import tpu_sc as plsc

_E = 320000
_N = 10000
_NU = 128
_NB = 16

_NC = 2
_NS = 16
_NW = _NC * _NS
_EPW = _E // _NW

_BN = 2000
_NBLK = _N // _BN


def _sc_scatter_body(idx1, idx2, vx, vy, part,
                     i1_v, i2_v, vx_v, vy_v, zbuf, acc, lsem, ssem):
    c = lax.axis_index("c")
    s = lax.axis_index("s")
    w = s * _NC + c
    base = w * _EPW

    @pl.when(s < _NBLK)
    def _zero():
        def zstore(i, carry):
            zbuf[pl.ds(i * 16, 16)] = jnp.zeros((16,), jnp.float32)
            return carry

        lax.fori_loop(0, _BN // 16, zstore, 0)
        pltpu.sync_copy(zbuf, acc.at[pl.ds(s * _BN, _BN)])

    loads = [
        pltpu.async_copy(idx1.at[pl.ds(base, _EPW)], i1_v, lsem),
        pltpu.async_copy(idx2.at[pl.ds(base, _EPW)], i2_v, lsem),
        pltpu.async_copy(vx.at[pl.ds(base, _EPW)], vx_v, lsem),
        pltpu.async_copy(vy.at[pl.ds(base, _EPW)], vy_v, lsem),
    ]
    for d in loads:
        d.wait()

    plsc.subcore_barrier()

    d1 = pltpu.async_copy(vx_v, acc.at[i1_v], ssem, add=True)
    d2 = pltpu.async_copy(vy_v, acc.at[i2_v], ssem, add=True)
    d1.wait()
    d2.wait()

    plsc.subcore_barrier()

    @pl.when(s < _NBLK)
    def _flush():
        pltpu.sync_copy(acc.at[pl.ds(s * _BN, _BN)], part.at[c, s, 0])


_sc_scatter = functools.partial(
    pl.kernel,
    out_type=jax.ShapeDtypeStruct((_NC, _NBLK, 1, _BN), jnp.float32),
    mesh=plsc.VectorSubcoreMesh(core_axis_name="c", subcore_axis_name="s",
                                num_cores=_NC, num_subcores=_NS),
    scratch_types=[
        pltpu.VMEM((_EPW,), jnp.int32),
        pltpu.VMEM((_EPW,), jnp.int32),
        pltpu.VMEM((_EPW,), jnp.float32),
        pltpu.VMEM((_EPW,), jnp.float32),
        pltpu.VMEM((_BN,), jnp.float32),
        pltpu.VMEM_SHARED((_N,), jnp.float32),
        pltpu.SemaphoreType.DMA,
        pltpu.SemaphoreType.DMA,
    ],
    compiler_params=pltpu.CompilerParams(use_tc_tiling_on_sc=False),
)(_sc_scatter_body)


def _tc_assemble_body(part_ref, out_ref):
    onehot = (lax.broadcasted_iota(jnp.int32, (_NC, _NU), 1) == 0)
    out_ref[...] = lax.dot_general(
        part_ref[:, 0, 0, :],
        onehot.astype(jnp.float32),
        (((0,), (0,)), ((), ())),
        preferred_element_type=jnp.float32,
    )


def kernel(unary, binary, deltas, index1, index2):
    del unary, binary

    i1 = index1.astype(jnp.int32)
    i2 = index2.astype(jnp.int32)
    vx = deltas[:, 0]
    vy = deltas[:, _NU]

    parts = _sc_scatter(i1, i2, vx, vy)

    out1 = pl.pallas_call(
        _tc_assemble_body,
        grid=(_NBLK,),
        in_specs=[pl.BlockSpec((_NC, 1, 1, _BN), lambda i: (0, i, 0, 0))],
        out_specs=pl.BlockSpec((_BN, _NU), lambda i: (i, 0)),
        out_shape=jax.ShapeDtypeStruct((_N, _NU), jnp.float32),
    )(parts)

    return (out1, deltas[:, 2 * _NU:])


# speedup vs baseline: 12.9362x; 1.0040x over previous
import functools

import jax
import jax.numpy as jnp
from jax import lax
from jax.experimental import pallas as pl
from jax.experimental.pallas import tpu as pltpu
from jax.experimental.pallas import tpu_sc as plsc

_E = 320000
_N = 10000
_NU = 128
_NB = 16

_NC = 2
_NS = 16
_NW = _NC * _NS
_EPW = _E // _NW

_BN = 2000
_NBLK = _N // _BN


def _sc_scatter_body(idx1, idx2, vx, vy, part,
                     i1_v, i2_v, vx_v, vy_v, zbuf, acc, lsem, ssem):
    c = lax.axis_index("c")
    s = lax.axis_index("s")
    w = s * _NC + c
    base = w * _EPW

    @pl.when(s < _NBLK)
    def _zero():
        def zstore(i, carry):
            zbuf[pl.ds(i * 16, 16)] = jnp.zeros((16,), jnp.float32)
            return carry

        lax.fori_loop(0, _BN // 16, zstore, 0)
        pltpu.sync_copy(zbuf, acc.at[pl.ds(s * _BN, _BN)])

    loads = [
        pltpu.async_copy(idx1.at[pl.ds(base, _EPW)], i1_v, lsem),
        pltpu.async_copy(idx2.at[pl.ds(base, _EPW)], i2_v, lsem),
        pltpu.async_copy(vx.at[pl.ds(base, _EPW)], vx_v, lsem),
        pltpu.async_copy(vy.at[pl.ds(base, _EPW)], vy_v, lsem),
    ]
    for d in loads:
        d.wait()

    plsc.subcore_barrier()

    d1 = pltpu.async_copy(vx_v, acc.at[i1_v], ssem, add=True)
    d2 = pltpu.async_copy(vy_v, acc.at[i2_v], ssem, add=True)
    d1.wait()
    d2.wait()

    plsc.subcore_barrier()

    @pl.when(s < _NBLK)
    def _flush():
        pltpu.sync_copy(acc.at[pl.ds(s * _BN, _BN)], part.at[c, s, 0])


_sc_scatter = functools.partial(
    pl.kernel,
    out_type=jax.ShapeDtypeStruct((_NC, _NBLK, 1, _BN), jnp.float32),
    mesh=plsc.VectorSubcoreMesh(core_axis_name="c", subcore_axis_name="s",
                                num_cores=_NC, num_subcores=_NS),
    scratch_types=[
        pltpu.VMEM((_EPW,), jnp.int32),
        pltpu.VMEM((_EPW,), jnp.int32),
        pltpu.VMEM((_EPW,), jnp.float32),
        pltpu.VMEM((_EPW,), jnp.float32),
        pltpu.VMEM((_BN,), jnp.float32),
        pltpu.VMEM_SHARED((_N,), jnp.float32),
        pltpu.SemaphoreType.DMA,
        pltpu.SemaphoreType.DMA,
    ],
    compiler_params=pltpu.CompilerParams(use_tc_tiling_on_sc=False),
)(_sc_scatter_body)


def _tc_assemble_body(part_ref, out_ref):
    onehot = (lax.broadcasted_iota(jnp.int32, (_NC, _NU), 1) == 0)
    out_ref[...] = lax.dot_general(
        part_ref[:, 0, 0, :],
        onehot.astype(jnp.float32),
        (((0,), (0,)), ((), ())),
        preferred_element_type=jnp.float32,
    )


def kernel(unary, binary, deltas, index1, index2):
    del unary, binary

    i1 = index1.astype(jnp.int32)
    i2 = index2.astype(jnp.int32)
    vx = deltas[:, 0]
    vy = deltas[:, _NU]

    parts = _sc_scatter(i1, i2, vx, vy)
    p = (parts[0] + parts[1]).reshape(_N, 1)
    out1 = jnp.pad(p, ((0, 0), (0, _NU - 1)))

    return (out1, deltas[:, 2 * _NU:])
